# baseline (speedup 1.0000x reference)
import functools

import jax
import jax.numpy as jnp
from jax.experimental import pallas as pl
from jax.experimental.pallas import tpu as pltpu

_T = 16384
_D = 2048
_E = 16
_BT = 512


def _router_body(x_ref, w_ref, w_out_ref, e_out_ref):
    logits = jnp.dot(x_ref[...], w_ref[...], preferred_element_type=jnp.float32)
    col = jax.lax.broadcasted_iota(jnp.int32, logits.shape, 1)
    m1 = jnp.max(logits, axis=1, keepdims=True)
    a1 = jnp.argmax(logits, axis=1)[:, None]
    masked = jnp.where(col == a1, -jnp.inf, logits)
    m2 = jnp.max(masked, axis=1, keepdims=True)
    a2 = jnp.argmax(masked, axis=1)[:, None]
    e = jnp.exp(m2 - m1)
    s = 1.0 / (1.0 + e)
    w_out_ref[...] = jnp.concatenate([s, e * s], axis=1)
    e_out_ref[...] = jnp.concatenate([a1, a2], axis=1)


@jax.jit
def kernel(x_TD, kernel_DE):
    x_TD = jnp.asarray(x_TD, jnp.float32)
    grid = (_T // _BT,)
    weights, experts = pl.pallas_call(
        _router_body,
        grid=grid,
        in_specs=[
            pl.BlockSpec((_BT, _D), lambda i: (i, 0)),
            pl.BlockSpec((_D, _E), lambda i: (0, 0)),
        ],
        out_specs=[
            pl.BlockSpec((_BT, 2), lambda i: (i, 0)),
            pl.BlockSpec((_BT, 2), lambda i: (i, 0)),
        ],
        out_shape=[
            jax.ShapeDtypeStruct((_T, 2), jnp.float32),
            jax.ShapeDtypeStruct((_T, 2), jnp.int32),
        ],
    )(x_TD, kernel_DE)
    return (weights, experts)


# speedup vs baseline: 1.1365x; 1.1365x over previous
import functools

import jax
import jax.numpy as jnp
from jax.experimental import pallas as pl
from jax.experimental.pallas import tpu as pltpu

_T = 16384
_D = 2048
_E = 16
_BT = 2048


def _router_body(x_ref, w_ref, w_out_ref, e_out_ref):
    logits = jnp.dot(x_ref[...], w_ref[...], preferred_element_type=jnp.float32)
    col = jax.lax.broadcasted_iota(jnp.int32, logits.shape, 1)
    m1 = jnp.max(logits, axis=1, keepdims=True)
    a1 = jnp.argmax(logits, axis=1)[:, None]
    masked = jnp.where(col == a1, -jnp.inf, logits)
    m2 = jnp.max(masked, axis=1, keepdims=True)
    a2 = jnp.argmax(masked, axis=1)[:, None]
    e = jnp.exp(m2 - m1)
    s = 1.0 / (1.0 + e)
    w_out_ref[...] = jnp.concatenate([s, e * s], axis=1)
    e_out_ref[...] = jnp.concatenate([a1, a2], axis=1)


@jax.jit
def kernel(x_TD, kernel_DE):
    x_TD = jnp.asarray(x_TD, jnp.float32)
    grid = (_T // _BT,)
    weights, experts = pl.pallas_call(
        _router_body,
        grid=grid,
        in_specs=[
            pl.BlockSpec((_BT, _D), lambda i: (i, 0)),
            pl.BlockSpec((_D, _E), lambda i: (0, 0)),
        ],
        out_specs=[
            pl.BlockSpec((_BT, 2), lambda i: (i, 0)),
            pl.BlockSpec((_BT, 2), lambda i: (i, 0)),
        ],
        out_shape=[
            jax.ShapeDtypeStruct((_T, 2), jnp.float32),
            jax.ShapeDtypeStruct((_T, 2), jnp.int32),
        ],
    )(x_TD, kernel_DE)
    return (weights, experts)


# speedup vs baseline: 1.1415x; 1.0044x over previous
import functools

import jax
import jax.numpy as jnp
from jax.experimental import pallas as pl
from jax.experimental.pallas import tpu as pltpu

_T = 16384
_D = 2048
_E = 16
_BT = 2048


def _router_body(x0_ref, x1_ref, x2_ref, x3_ref, w_ref, w_out_ref, e_out_ref):
    dc = _D // 4
    logits = (
        jnp.dot(x0_ref[...], w_ref[0 * dc : 1 * dc, :], preferred_element_type=jnp.float32)
        + jnp.dot(x1_ref[...], w_ref[1 * dc : 2 * dc, :], preferred_element_type=jnp.float32)
        + jnp.dot(x2_ref[...], w_ref[2 * dc : 3 * dc, :], preferred_element_type=jnp.float32)
        + jnp.dot(x3_ref[...], w_ref[3 * dc : 4 * dc, :], preferred_element_type=jnp.float32)
    )
    col = jax.lax.broadcasted_iota(jnp.int32, logits.shape, 1)
    m1 = jnp.max(logits, axis=1, keepdims=True)
    a1 = jnp.argmax(logits, axis=1)[:, None]
    masked = jnp.where(col == a1, -jnp.inf, logits)
    m2 = jnp.max(masked, axis=1, keepdims=True)
    a2 = jnp.argmax(masked, axis=1)[:, None]
    e = jnp.exp(m2 - m1)
    s = 1.0 / (1.0 + e)
    w_out_ref[...] = jnp.concatenate([s, e * s], axis=1)
    e_out_ref[...] = jnp.concatenate([a1, a2], axis=1)


@jax.jit
def kernel(x_TD, kernel_DE):
    x_TD = jnp.asarray(x_TD, jnp.float32)
    grid = (_T // _BT,)
    weights, experts = pl.pallas_call(
        _router_body,
        grid=grid,
        in_specs=[
            pl.BlockSpec((_BT, _D // 4), lambda i: (i, 0)),
            pl.BlockSpec((_BT, _D // 4), lambda i: (i, 1)),
            pl.BlockSpec((_BT, _D // 4), lambda i: (i, 2)),
            pl.BlockSpec((_BT, _D // 4), lambda i: (i, 3)),
            pl.BlockSpec((_D, _E), lambda i: (0, 0)),
        ],
        out_specs=[
            pl.BlockSpec((_BT, 2), lambda i: (i, 0)),
            pl.BlockSpec((_BT, 2), lambda i: (i, 0)),
        ],
        out_shape=[
            jax.ShapeDtypeStruct((_T, 2), jnp.float32),
            jax.ShapeDtypeStruct((_T, 2), jnp.int32),
        ],
    )(x_TD, x_TD, x_TD, x_TD, kernel_DE)
    return (weights, experts)
